# baseline (speedup 1.0000x reference)
import functools

import jax
import jax.numpy as jnp
from jax import lax
from jax.experimental import pallas as pl
from jax.experimental.pallas import tpu as pltpu
from jax.experimental.pallas import tpu_sc as plsc

N = 10000
E = 320000
D = 128

NC = 2
NS = 16
NW = NC * NS
EPW = E // NW
C = 40
NCHUNK = EPW // C
G = 10
SG = NCHUNK // G
NPAD = 10240
RPT = NPAD // NS

MBLK = 1000
GRID = N // MBLK


def _proj_body(x_ref, w_ref, b_ref, kn_ref, qv_ref, s_ref):
    y = jnp.dot(x_ref[...], w_ref[...], preferred_element_type=jnp.float32)
    y = y + b_ref[0:1, :]
    kn_ref[...] = y[:, 0 * D:1 * D]
    qv_ref[...] = y[:, 1 * D:3 * D]
    s_ref[...] = y[:, 3 * D:4 * D]


def _relu_proj_body(p0_ref, p1_ref, s1_ref, w_ref, b_ref,
                    kn_ref, qv_ref, s_ref):
    h = jnp.maximum(p0_ref[...] + p1_ref[...] + s1_ref[...], 0.0)
    y = jnp.dot(h, w_ref[...], preferred_element_type=jnp.float32)
    y = y + b_ref[0:1, :]
    kn_ref[...] = y[:, 0 * D:1 * D]
    qv_ref[...] = y[:, 1 * D:3 * D]
    s_ref[...] = y[:, 3 * D:4 * D]


def _final_body(p0_ref, p1_ref, s2_ref, o_ref):
    o_ref[...] = p0_ref[...] + p1_ref[...] + s2_ref[...]


_row_spec = pl.BlockSpec((MBLK, D), lambda i: (i, 0))
_qv_spec = pl.BlockSpec((MBLK, 2 * D), lambda i: (i, 0))
_w_spec = pl.BlockSpec((D, 4 * D), lambda i: (0, 0))
_b_spec = pl.BlockSpec((8, 4 * D), lambda i: (0, 0))
_out3 = (jax.ShapeDtypeStruct((N, D), jnp.float32),
         jax.ShapeDtypeStruct((N, 2 * D), jnp.float32),
         jax.ShapeDtypeStruct((N, D), jnp.float32))

_proj = pl.pallas_call(
    _proj_body,
    grid=(GRID,),
    in_specs=[_row_spec, _w_spec, _b_spec],
    out_specs=(_row_spec, _qv_spec, _row_spec),
    out_shape=_out3,
)

_relu_proj = pl.pallas_call(
    _relu_proj_body,
    grid=(GRID,),
    in_specs=[_row_spec, _row_spec, _row_spec, _w_spec, _b_spec],
    out_specs=(_row_spec, _qv_spec, _row_spec),
    out_shape=_out3,
)

_final = pl.pallas_call(
    _final_body,
    grid=(GRID,),
    in_specs=[_row_spec, _row_spec, _row_spec],
    out_specs=_row_spec,
    out_shape=jax.ShapeDtypeStruct((N, D), jnp.float32),
)


_mesh = plsc.VectorSubcoreMesh(
    core_axis_name="c", subcore_axis_name="s", num_cores=NC, num_subcores=NS)


@functools.partial(
    pl.kernel,
    out_type=jax.ShapeDtypeStruct((NC, NPAD, D), jnp.float32),
    mesh=_mesh,
    scratch_types=[
        [pltpu.VMEM((G, 2, C), jnp.int32)] * 2,
        [pltpu.VMEM((C, D), jnp.float32)] * 2,
        [pltpu.VMEM((C, 2 * D), jnp.float32)] * 2,
        [pltpu.VMEM((C, D), jnp.float32)] * 2,
        pltpu.VMEM_SHARED((NPAD, D), jnp.float32),
        [pltpu.SemaphoreType.DMA] * 2,
        [pltpu.SemaphoreType.DMA] * 2,
        pltpu.SemaphoreType.DMA,
    ],
)
def _edge_kernel(kn_hbm, qv_hbm, sd_hbm, out_hbm,
                 idxv, kdv, qvv, mv, agg, gsem, ssem, isem):
    c = lax.axis_index("c")
    s = lax.axis_index("s")
    wid = s * NC + c

    def _zero_row(r, carry):
        for j in range(D // 16):
            kdv[0][r, pl.ds(j * 16, 16)] = jnp.zeros((16,), jnp.float32)
        return carry
    lax.fori_loop(0, C, _zero_row, 0)
    for t in range(RPT // C):
        pltpu.async_copy(kdv[0], agg.at[pl.ds(s * RPT + t * C, C)], gsem[0])
    for t in range(RPT // C):
        pltpu.make_async_copy(kdv[0], agg.at[pl.ds(s * RPT, C)], gsem[0]).wait()
    plsc.subcore_barrier()

    def _gather(g, b, sb):
        pltpu.async_copy(kn_hbm.at[idxv[sb].at[g, 1]], kdv[b], gsem[b])
        pltpu.async_copy(qv_hbm.at[idxv[sb].at[g, 0]], qvv[b], gsem[b])

    def _wait_gather(b):
        i0 = idxv[0].at[0, 0]
        pltpu.make_async_copy(kn_hbm.at[i0], kdv[b], gsem[b]).wait()
        pltpu.make_async_copy(qv_hbm.at[i0], qvv[b], gsem[b]).wait()

    def _wait_scatter(b):
        pltpu.make_async_copy(mv[b], agg.at[idxv[0].at[0, 1]], ssem[b]).wait()

    def _step(g, b, sb, wait_scat, gather_next):
        if gather_next:
            _gather(g + 1, 1 - b, sb)
        _wait_gather(b)
        if wait_scat:
            _wait_scatter(b)

        def _row(r, rc):
            for j in range(D // 16):
                sl = pl.ds(j * 16, 16)
                z = kdv[b][r, sl] + qvv[b][r, sl]
                mv[b][r, sl] = qvv[b][r, pl.ds(D + j * 16, 16)] / (
                    1.0 + jnp.exp(z))
            return rc
        lax.fori_loop(0, C, _row, 0)
        pltpu.async_copy(mv[b], agg.at[idxv[sb].at[g, 1]], ssem[b], add=True)

    def _super(sc, sb, first):
        if not first:
            pltpu.make_async_copy(sd_hbm.at[wid, 0], idxv[sb], isem).wait()
            _wait_scatter(0)
            _wait_scatter(1)
        pltpu.async_copy(sd_hbm.at[wid, sc + 1], idxv[1 - sb], isem)
        _gather(0, 0, sb)
        _step(0, 0, sb, False, True)
        _step(1, 1, sb, False, True)

        def _pairs(p, cc):
            g = 2 * p
            _step(g, 0, sb, True, True)
            _step(g + 1, 1, sb, True, True)
            return cc
        lax.fori_loop(1, G // 2 - 1, _pairs, 0)
        _step(G - 2, 0, sb, True, True)
        _step(G - 1, 1, sb, True, False)

    pltpu.sync_copy(sd_hbm.at[wid, 0], idxv[0])
    _super(0, 0, True)

    def _souter(p, cc):
        _super(2 * p + 1, 1, False)
        _super(2 * p + 2, 0, False)
        return cc
    lax.fori_loop(0, (SG - 1) // 2, _souter, 0)
    pltpu.make_async_copy(sd_hbm.at[wid, 0], idxv[1], isem).wait()
    _wait_scatter(0)
    _wait_scatter(1)

    plsc.subcore_barrier()
    pltpu.sync_copy(agg.at[pl.ds(s * RPT, RPT)],
                    out_hbm.at[c, pl.ds(s * RPT, RPT)])


def kernel(x, edge_index, Wk1, bk1, Wq1, bq1, Wv1, bv1, Ws1, b1,
           Wk2, bk2, Wq2, bq2, Wv2, bv2, Ws2, b2):
    src = edge_index[0].reshape(NW, SG, G, 1, C)
    dst = edge_index[1].reshape(NW, SG, G, 1, C)
    sd = jnp.concatenate([src, dst], axis=3)
    sd = jnp.concatenate(
        [sd, jnp.zeros((NW, 1, G, 2, C), jnp.int32)], axis=1)

    w1 = jnp.concatenate([-Wk1, -Wq1, Wv1, Ws1], axis=1)
    b1c = jnp.broadcast_to(
        jnp.concatenate([-bk1, -bq1, bv1, b1])[None, :], (8, 4 * D))
    w2 = jnp.concatenate([-Wk2, -Wq2, Wv2, Ws2], axis=1)
    b2c = jnp.broadcast_to(
        jnp.concatenate([-bk2, -bq2, bv2, b2])[None, :], (8, 4 * D))

    kn1, qv1, s1 = _proj(x, w1, b1c)
    part1 = _edge_kernel(kn1, qv1, sd)
    kn2, qv2, s2 = _relu_proj(part1[0, :N], part1[1, :N], s1, w2, b2c)
    part2 = _edge_kernel(kn2, qv2, sd)
    return _final(part2[0, :N], part2[1, :N], s2)


# speedup vs baseline: 1.0156x; 1.0156x over previous
import functools

import jax
import jax.numpy as jnp
from jax import lax
from jax.experimental import pallas as pl
from jax.experimental.pallas import tpu as pltpu
from jax.experimental.pallas import tpu_sc as plsc

N = 10000
E = 320000
D = 128

NC = 2
NS = 16
NW = NC * NS
EPW = E // NW
C = 40
NCHUNK = EPW // C
G = 10
SG = NCHUNK // G
NPAD = 10240
RPT = NPAD // NS

MBLK = 1000
GRID = N // MBLK


def _proj_body(x_ref, w_ref, b_ref, kn_ref, qv_ref, s_ref):
    y = jnp.dot(x_ref[...], w_ref[...], preferred_element_type=jnp.float32)
    y = y + b_ref[0:1, :]
    kn_ref[...] = y[:, 0 * D:1 * D]
    qv_ref[...] = y[:, 1 * D:3 * D]
    s_ref[...] = y[:, 3 * D:4 * D]


def _relu_proj_body(p0_ref, p1_ref, s1_ref, w_ref, b_ref,
                    kn_ref, qv_ref, s_ref):
    h = jnp.maximum(p0_ref[...] + p1_ref[...] + s1_ref[...], 0.0)
    y = jnp.dot(h, w_ref[...], preferred_element_type=jnp.float32)
    y = y + b_ref[0:1, :]
    kn_ref[...] = y[:, 0 * D:1 * D]
    qv_ref[...] = y[:, 1 * D:3 * D]
    s_ref[...] = y[:, 3 * D:4 * D]


def _final_body(p0_ref, p1_ref, s2_ref, o_ref):
    o_ref[...] = p0_ref[...] + p1_ref[...] + s2_ref[...]


_row_spec = pl.BlockSpec((MBLK, D), lambda i: (i, 0))
_qv_spec = pl.BlockSpec((MBLK, 2 * D), lambda i: (i, 0))
_w_spec = pl.BlockSpec((D, 4 * D), lambda i: (0, 0))
_b_spec = pl.BlockSpec((8, 4 * D), lambda i: (0, 0))
_out3 = (jax.ShapeDtypeStruct((N, D), jnp.float32),
         jax.ShapeDtypeStruct((N, 2 * D), jnp.float32),
         jax.ShapeDtypeStruct((N, D), jnp.float32))

_proj = pl.pallas_call(
    _proj_body,
    grid=(GRID,),
    in_specs=[_row_spec, _w_spec, _b_spec],
    out_specs=(_row_spec, _qv_spec, _row_spec),
    out_shape=_out3,
)

_relu_proj = pl.pallas_call(
    _relu_proj_body,
    grid=(GRID,),
    in_specs=[_row_spec, _row_spec, _row_spec, _w_spec, _b_spec],
    out_specs=(_row_spec, _qv_spec, _row_spec),
    out_shape=_out3,
)

_final = pl.pallas_call(
    _final_body,
    grid=(GRID,),
    in_specs=[_row_spec, _row_spec, _row_spec],
    out_specs=_row_spec,
    out_shape=jax.ShapeDtypeStruct((N, D), jnp.float32),
)


_mesh = plsc.VectorSubcoreMesh(
    core_axis_name="c", subcore_axis_name="s", num_cores=NC, num_subcores=NS)


@functools.partial(
    pl.kernel,
    out_type=jax.ShapeDtypeStruct((NC, NPAD, D), jnp.float32),
    mesh=_mesh,
    scratch_types=[
        [pltpu.VMEM((G, 2, C), jnp.int32)] * 2,
        [pltpu.VMEM((C, D), jnp.float32)] * 2,
        [pltpu.VMEM((C, 2, D), jnp.float32)] * 2,
        [pltpu.VMEM((C, D), jnp.float32)] * 2,
        pltpu.VMEM_SHARED((NPAD, D), jnp.float32),
        [pltpu.SemaphoreType.DMA] * 2,
        [pltpu.SemaphoreType.DMA] * 2,
        pltpu.SemaphoreType.DMA,
    ],
)
def _edge_kernel(kn_hbm, qv_hbm, sd_hbm, out_hbm,
                 idxv, kdv, qvv, mv, agg, gsem, ssem, isem):
    c = lax.axis_index("c")
    s = lax.axis_index("s")
    wid = s * NC + c

    def _zero_row(r, carry):
        for j in range(D // 16):
            kdv[0][r, pl.ds(j * 16, 16)] = jnp.zeros((16,), jnp.float32)
        return carry
    lax.fori_loop(0, C, _zero_row, 0)
    for t in range(RPT // C):
        pltpu.async_copy(kdv[0], agg.at[pl.ds(s * RPT + t * C, C)], gsem[0])
    for t in range(RPT // C):
        pltpu.make_async_copy(kdv[0], agg.at[pl.ds(s * RPT, C)], gsem[0]).wait()
    plsc.subcore_barrier()

    def _gather(g, b, sb):
        pltpu.async_copy(kn_hbm.at[idxv[sb].at[g, 1]], kdv[b], gsem[b])
        pltpu.async_copy(qv_hbm.at[idxv[sb].at[g, 0]], qvv[b], gsem[b])

    def _wait_gather(b):
        i0 = idxv[0].at[0, 0]
        pltpu.make_async_copy(kn_hbm.at[i0], kdv[b], gsem[b]).wait()
        pltpu.make_async_copy(qv_hbm.at[i0], qvv[b], gsem[b]).wait()

    def _wait_scatter(b):
        pltpu.make_async_copy(mv[b], agg.at[idxv[0].at[0, 1]], ssem[b]).wait()

    def _step(g, b, sb, wait_scat, gather_next):
        if gather_next:
            _gather(g + 1, 1 - b, sb)
        _wait_gather(b)
        if wait_scat:
            _wait_scatter(b)

        def _row(r, rc):
            for j in range(D // 16):
                sl = pl.ds(j * 16, 16)
                z = kdv[b][r, sl] + qvv[b][r, 0, sl]
                mv[b][r, sl] = qvv[b][r, 1, sl] / (1.0 + jnp.exp(z))
            return rc
        lax.fori_loop(0, C, _row, 0)
        pltpu.async_copy(mv[b], agg.at[idxv[sb].at[g, 1]], ssem[b], add=True)

    def _super(sc, sb, first):
        if not first:
            pltpu.make_async_copy(sd_hbm.at[wid, 0], idxv[sb], isem).wait()
            _wait_scatter(0)
            _wait_scatter(1)
        pltpu.async_copy(sd_hbm.at[wid, sc + 1], idxv[1 - sb], isem)
        _gather(0, 0, sb)
        _step(0, 0, sb, False, True)
        _step(1, 1, sb, False, True)

        def _pairs(p, cc):
            g = 2 * p
            _step(g, 0, sb, True, True)
            _step(g + 1, 1, sb, True, True)
            return cc
        lax.fori_loop(1, G // 2 - 1, _pairs, 0)
        _step(G - 2, 0, sb, True, True)
        _step(G - 1, 1, sb, True, False)

    pltpu.sync_copy(sd_hbm.at[wid, 0], idxv[0])
    _super(0, 0, True)

    def _souter(p, cc):
        _super(2 * p + 1, 1, False)
        _super(2 * p + 2, 0, False)
        return cc
    lax.fori_loop(0, (SG - 1) // 2, _souter, 0)
    pltpu.make_async_copy(sd_hbm.at[wid, 0], idxv[1], isem).wait()
    _wait_scatter(0)
    _wait_scatter(1)

    plsc.subcore_barrier()
    pltpu.sync_copy(agg.at[pl.ds(s * RPT, RPT)],
                    out_hbm.at[c, pl.ds(s * RPT, RPT)])


def kernel(x, edge_index, Wk1, bk1, Wq1, bq1, Wv1, bv1, Ws1, b1,
           Wk2, bk2, Wq2, bq2, Wv2, bv2, Ws2, b2):
    src = edge_index[0].reshape(NW, SG, G, 1, C)
    dst = edge_index[1].reshape(NW, SG, G, 1, C)
    sd = jnp.concatenate([src, dst], axis=3)
    sd = jnp.concatenate(
        [sd, jnp.zeros((NW, 1, G, 2, C), jnp.int32)], axis=1)

    w1 = jnp.concatenate([-Wk1, -Wq1, Wv1, Ws1], axis=1)
    b1c = jnp.broadcast_to(
        jnp.concatenate([-bk1, -bq1, bv1, b1])[None, :], (8, 4 * D))
    w2 = jnp.concatenate([-Wk2, -Wq2, Wv2, Ws2], axis=1)
    b2c = jnp.broadcast_to(
        jnp.concatenate([-bk2, -bq2, bv2, b2])[None, :], (8, 4 * D))

    kn1, qv1, s1 = _proj(x, w1, b1c)
    part1 = _edge_kernel(kn1, qv1.reshape(N, 2, D), sd)
    kn2, qv2, s2 = _relu_proj(part1[0, :N], part1[1, :N], s1, w2, b2c)
    part2 = _edge_kernel(kn2, qv2.reshape(N, 2, D), sd)
    return _final(part2[0, :N], part2[1, :N], s2)


# speedup vs baseline: 4.7551x; 4.6821x over previous
import functools

import jax
import jax.numpy as jnp
import numpy as np
from jax import lax
from jax.experimental import pallas as pl
from jax.experimental.pallas import tpu as pltpu
from jax.experimental.pallas import tpu_sc as plsc

N = 10000
E = 320000
D = 128

NC = 2
NS = 16
NW = NC * NS
EPW = E // NW
C = 40
NCHUNK = EPW // C
G = 10
SG = NCHUNK // G
NPAD = 10240
RPT = NPAD // NS

MBLK = 1000
GRID = N // MBLK

_cols = np.arange(D)
_g, _w = _cols // 32, _cols % 32
PERM = tuple((_g * 32 + np.where(_w % 2 == 0, _w // 2, 16 + _w // 2)).tolist())


def _proj_body(x_ref, w_ref, b_ref, ek_ref, eq_ref, v_ref, s_ref):
    y = jnp.dot(x_ref[...], w_ref[...], preferred_element_type=jnp.float32)
    y = y + b_ref[0:1, :]
    ek_ref[...] = jnp.exp(y[:, 0 * D:1 * D])
    eq_ref[...] = jnp.exp(y[:, 1 * D:2 * D]).astype(jnp.bfloat16)
    v_ref[...] = y[:, 2 * D:3 * D].astype(jnp.bfloat16)
    s_ref[...] = y[:, 3 * D:4 * D]


def _relu_proj_body(p0_ref, p1_ref, s1_ref, w_ref, b_ref,
                    ek_ref, eq_ref, v_ref, s_ref):
    h = jnp.maximum(p0_ref[...] + p1_ref[...] + s1_ref[...], 0.0)
    y = jnp.dot(h, w_ref[...], preferred_element_type=jnp.float32)
    y = y + b_ref[0:1, :]
    ek_ref[...] = jnp.exp(y[:, 0 * D:1 * D])
    eq_ref[...] = jnp.exp(y[:, 1 * D:2 * D]).astype(jnp.bfloat16)
    v_ref[...] = y[:, 2 * D:3 * D].astype(jnp.bfloat16)
    s_ref[...] = y[:, 3 * D:4 * D]


def _final_body(p0_ref, p1_ref, s2_ref, o_ref):
    o_ref[...] = p0_ref[...] + p1_ref[...] + s2_ref[...]


_row_spec = pl.BlockSpec((MBLK, D), lambda i: (i, 0))
_w_spec = pl.BlockSpec((D, 4 * D), lambda i: (0, 0))
_b_spec = pl.BlockSpec((8, 4 * D), lambda i: (0, 0))
_out4 = (jax.ShapeDtypeStruct((N, D), jnp.float32),
         jax.ShapeDtypeStruct((N, D), jnp.bfloat16),
         jax.ShapeDtypeStruct((N, D), jnp.bfloat16),
         jax.ShapeDtypeStruct((N, D), jnp.float32))

_proj = pl.pallas_call(
    _proj_body,
    grid=(GRID,),
    in_specs=[_row_spec, _w_spec, _b_spec],
    out_specs=(_row_spec,) * 4,
    out_shape=_out4,
)

_relu_proj = pl.pallas_call(
    _relu_proj_body,
    grid=(GRID,),
    in_specs=[_row_spec, _row_spec, _row_spec, _w_spec, _b_spec],
    out_specs=(_row_spec,) * 4,
    out_shape=_out4,
)

_final = pl.pallas_call(
    _final_body,
    grid=(GRID,),
    in_specs=[_row_spec, _row_spec, _row_spec],
    out_specs=_row_spec,
    out_shape=jax.ShapeDtypeStruct((N, D), jnp.float32),
)


_mesh = plsc.VectorSubcoreMesh(
    core_axis_name="c", subcore_axis_name="s", num_cores=NC, num_subcores=NS)


@functools.partial(
    pl.kernel,
    out_type=jax.ShapeDtypeStruct((NC, NPAD, D), jnp.float32),
    mesh=_mesh,
    scratch_types=[
        [pltpu.VMEM((G, 2, C), jnp.int32)] * 2,
        [pltpu.VMEM((C, D), jnp.float32)] * 2,
        [pltpu.VMEM((C, D), jnp.int32)] * 2,
        [pltpu.VMEM((C, D), jnp.float32)] * 2,
        pltpu.VMEM_SHARED((NPAD, D), jnp.float32),
        [pltpu.SemaphoreType.DMA] * 2,
        [pltpu.SemaphoreType.DMA] * 2,
        pltpu.SemaphoreType.DMA,
    ],
)
def _edge_kernel(ek_hbm, qv_hbm, sd_hbm, out_hbm,
                 idxv, ekv, qvv, mv, agg, gsem, ssem, isem):
    c = lax.axis_index("c")
    s = lax.axis_index("s")
    wid = s * NC + c

    def _zero_row(r, carry):
        for j in range(D // 16):
            mv[0][r, pl.ds(j * 16, 16)] = jnp.zeros((16,), jnp.float32)
        return carry
    lax.fori_loop(0, C, _zero_row, 0)
    for t in range(RPT // C):
        pltpu.async_copy(mv[0], agg.at[pl.ds(s * RPT + t * C, C)], gsem[0])
    for t in range(RPT // C):
        pltpu.make_async_copy(mv[0], agg.at[pl.ds(s * RPT, C)], gsem[0]).wait()
    plsc.subcore_barrier()

    def _gather(g, b, sb):
        pltpu.async_copy(ek_hbm.at[idxv[sb].at[g, 1]], ekv[b], gsem[b])
        pltpu.async_copy(qv_hbm.at[idxv[sb].at[g, 0]], qvv[b], gsem[b])

    def _wait_gather(b):
        i0 = idxv[0].at[0, 0]
        pltpu.make_async_copy(ek_hbm.at[i0], ekv[b], gsem[b]).wait()
        pltpu.make_async_copy(qv_hbm.at[i0], qvv[b], gsem[b]).wait()

    def _wait_scatter(b):
        pltpu.make_async_copy(mv[b], agg.at[idxv[0].at[0, 1]], ssem[b]).wait()

    def _step(g, b, sb, wait_scat, gather_next):
        if gather_next:
            _gather(g + 1, 1 - b, sb)
        _wait_gather(b)
        if wait_scat:
            _wait_scatter(b)

        def _cvt(w):
            lo = lax.bitcast_convert_type(w << 16, jnp.float32)
            hi = lax.bitcast_convert_type(w & jnp.int32(-65536), jnp.float32)
            return lo, hi

        def _row(r, rc):
            for j in range(D // 32):
                eq0, eq1 = _cvt(qvv[b][r, pl.ds(j * 16, 16)])
                v0, v1 = _cvt(qvv[b][r, pl.ds(64 + j * 16, 16)])
                ek0 = ekv[b][r, pl.ds(j * 32, 16)]
                ek1 = ekv[b][r, pl.ds(j * 32 + 16, 16)]
                mv[b][r, pl.ds(j * 32, 16)] = v0 / (1.0 + ek0 * eq0)
                mv[b][r, pl.ds(j * 32 + 16, 16)] = v1 / (1.0 + ek1 * eq1)
            return rc
        lax.fori_loop(0, C, _row, 0)
        pltpu.async_copy(mv[b], agg.at[idxv[sb].at[g, 1]], ssem[b], add=True)

    def _super(sc, sb, first):
        if not first:
            pltpu.make_async_copy(sd_hbm.at[wid, 0], idxv[sb], isem).wait()
            _wait_scatter(0)
            _wait_scatter(1)
        pltpu.async_copy(sd_hbm.at[wid, sc + 1], idxv[1 - sb], isem)
        _gather(0, 0, sb)
        _step(0, 0, sb, False, True)
        _step(1, 1, sb, False, True)

        def _pairs(p, cc):
            g = 2 * p
            _step(g, 0, sb, True, True)
            _step(g + 1, 1, sb, True, True)
            return cc
        lax.fori_loop(1, G // 2 - 1, _pairs, 0)
        _step(G - 2, 0, sb, True, True)
        _step(G - 1, 1, sb, True, False)

    pltpu.sync_copy(sd_hbm.at[wid, 0], idxv[0])
    _super(0, 0, True)

    def _souter(p, cc):
        _super(2 * p + 1, 1, False)
        _super(2 * p + 2, 0, False)
        return cc
    lax.fori_loop(0, (SG - 1) // 2, _souter, 0)
    pltpu.make_async_copy(sd_hbm.at[wid, 0], idxv[1], isem).wait()
    _wait_scatter(0)
    _wait_scatter(1)

    plsc.subcore_barrier()
    pltpu.sync_copy(agg.at[pl.ds(s * RPT, RPT)],
                    out_hbm.at[c, pl.ds(s * RPT, RPT)])


def kernel(x, edge_index, Wk1, bk1, Wq1, bq1, Wv1, bv1, Ws1, b1,
           Wk2, bk2, Wq2, bq2, Wv2, bv2, Ws2, b2):
    src = edge_index[0].reshape(NW, SG, G, 1, C)
    dst = edge_index[1].reshape(NW, SG, G, 1, C)
    sd = jnp.concatenate([src, dst], axis=3)
    sd = jnp.concatenate(
        [sd, jnp.zeros((NW, 1, G, 2, C), jnp.int32)], axis=1)

    perm = jnp.asarray(PERM)
    w1 = jnp.concatenate(
        [-Wk1, -Wq1[:, perm], Wv1[:, perm], Ws1], axis=1)
    b1c = jnp.broadcast_to(jnp.concatenate(
        [-bk1, -bq1[perm], bv1[perm], b1])[None, :], (8, 4 * D))
    w2 = jnp.concatenate(
        [-Wk2, -Wq2[:, perm], Wv2[:, perm], Ws2], axis=1)
    b2c = jnp.broadcast_to(jnp.concatenate(
        [-bk2, -bq2[perm], bv2[perm], b2])[None, :], (8, 4 * D))

    def _packqv(eq, v):
        pk = lambda t: lax.bitcast_convert_type(
            t.reshape(N, D // 2, 2), jnp.int32)
        return jnp.concatenate([pk(eq), pk(v)], axis=1)

    ek1, eq1, v1, s1 = _proj(x, w1, b1c)
    part1 = _edge_kernel(ek1, _packqv(eq1, v1), sd)
    ek2, eq2, v2, s2 = _relu_proj(part1[0, :N], part1[1, :N], s1, w2, b2c)
    part2 = _edge_kernel(ek2, _packqv(eq2, v2), sd)
    return _final(part2[0, :N], part2[1, :N], s2)


# speedup vs baseline: 5.0691x; 1.0660x over previous
import functools

import jax
import jax.numpy as jnp
import numpy as np
from jax import lax
from jax.experimental import pallas as pl
from jax.experimental.pallas import tpu as pltpu
from jax.experimental.pallas import tpu_sc as plsc

N = 10000
E = 320000
D = 128

NC = 2
NS = 16
NW = NC * NS
EPW = E // NW
C = 40
NCHUNK = EPW // C
G = 10
SG = NCHUNK // G
NPAD = 10240
RPT = NPAD // NS

MBLK = 1000
GRID = N // MBLK

_cols = np.arange(D)
_g, _w = _cols // 32, _cols % 32
PERM = tuple((_g * 32 + np.where(_w % 2 == 0, _w // 2, 16 + _w // 2)).tolist())


def _proj_body(x_ref, w_ref, b_ref, ek_ref, eq_ref, v_ref, s_ref):
    y = jnp.dot(x_ref[...], w_ref[...], preferred_element_type=jnp.float32)
    y = y + b_ref[0:1, :]
    ek_ref[...] = jnp.exp(y[:, 0 * D:1 * D])
    eq_ref[...] = jnp.exp(y[:, 1 * D:2 * D]).astype(jnp.bfloat16)
    v_ref[...] = y[:, 2 * D:3 * D].astype(jnp.bfloat16)
    s_ref[...] = y[:, 3 * D:4 * D]


def _relu_proj_body(p_ref, s1_ref, w_ref, b_ref,
                    ek_ref, eq_ref, v_ref, s_ref):
    h = jnp.maximum(p_ref[0] + p_ref[1] + s1_ref[...], 0.0)
    y = jnp.dot(h, w_ref[...], preferred_element_type=jnp.float32)
    y = y + b_ref[0:1, :]
    ek_ref[...] = jnp.exp(y[:, 0 * D:1 * D])
    eq_ref[...] = jnp.exp(y[:, 1 * D:2 * D]).astype(jnp.bfloat16)
    v_ref[...] = y[:, 2 * D:3 * D].astype(jnp.bfloat16)
    s_ref[...] = y[:, 3 * D:4 * D]


def _final_body(p_ref, s2_ref, o_ref):
    o_ref[...] = p_ref[0] + p_ref[1] + s2_ref[...]


_row_spec = pl.BlockSpec((MBLK, D), lambda i: (i, 0))
_part_spec = pl.BlockSpec((2, MBLK, D), lambda i: (0, i, 0))
_w_spec = pl.BlockSpec((D, 4 * D), lambda i: (0, 0))
_b_spec = pl.BlockSpec((8, 4 * D), lambda i: (0, 0))
_out4 = (jax.ShapeDtypeStruct((N, D), jnp.float32),
         jax.ShapeDtypeStruct((N, D), jnp.bfloat16),
         jax.ShapeDtypeStruct((N, D), jnp.bfloat16),
         jax.ShapeDtypeStruct((N, D), jnp.float32))

_proj = pl.pallas_call(
    _proj_body,
    grid=(GRID,),
    in_specs=[_row_spec, _w_spec, _b_spec],
    out_specs=(_row_spec,) * 4,
    out_shape=_out4,
)

_relu_proj = pl.pallas_call(
    _relu_proj_body,
    grid=(GRID,),
    in_specs=[_part_spec, _row_spec, _w_spec, _b_spec],
    out_specs=(_row_spec,) * 4,
    out_shape=_out4,
)

_final = pl.pallas_call(
    _final_body,
    grid=(GRID,),
    in_specs=[_part_spec, _row_spec],
    out_specs=_row_spec,
    out_shape=jax.ShapeDtypeStruct((N, D), jnp.float32),
)


_mesh = plsc.VectorSubcoreMesh(
    core_axis_name="c", subcore_axis_name="s", num_cores=NC, num_subcores=NS)


@functools.partial(
    pl.kernel,
    out_type=jax.ShapeDtypeStruct((NC, NPAD, D), jnp.float32),
    mesh=_mesh,
    scratch_types=[
        [pltpu.VMEM((2, G, C), jnp.int32)] * 2,
        [pltpu.VMEM((C, D), jnp.float32)] * 2,
        [pltpu.VMEM((C, D), jnp.int32)] * 2,
        [pltpu.VMEM((C, D), jnp.float32)] * 2,
        pltpu.VMEM_SHARED((NPAD, D), jnp.float32),
        [pltpu.SemaphoreType.DMA] * 2,
        [pltpu.SemaphoreType.DMA] * 2,
        pltpu.SemaphoreType.DMA,
    ],
)
def _edge_kernel(ek_hbm, qv_hbm, ei_hbm, out_hbm,
                 idxv, ekv, qvv, mv, agg, gsem, ssem, isem):
    c = lax.axis_index("c")
    s = lax.axis_index("s")
    wid = s * NC + c

    def _zero_row(r, carry):
        for j in range(D // 16):
            mv[0][r, pl.ds(j * 16, 16)] = jnp.zeros((16,), jnp.float32)
        return carry
    lax.fori_loop(0, C, _zero_row, 0)
    for t in range(RPT // C):
        pltpu.async_copy(mv[0], agg.at[pl.ds(s * RPT + t * C, C)], gsem[0])
    for t in range(RPT // C):
        pltpu.make_async_copy(mv[0], agg.at[pl.ds(s * RPT, C)], gsem[0]).wait()
    plsc.subcore_barrier()

    def _gather(g, b, sb):
        pltpu.async_copy(ek_hbm.at[idxv[sb].at[1, g]], ekv[b], gsem[b])
        pltpu.async_copy(qv_hbm.at[idxv[sb].at[0, g]], qvv[b], gsem[b])

    def _wait_gather(b):
        i0 = idxv[0].at[0, 0]
        pltpu.make_async_copy(ek_hbm.at[i0], ekv[b], gsem[b]).wait()
        pltpu.make_async_copy(qv_hbm.at[i0], qvv[b], gsem[b]).wait()

    def _wait_scatter(b):
        pltpu.make_async_copy(mv[b], agg.at[idxv[0].at[1, 0]], ssem[b]).wait()

    def _step(g, b, sb, wait_scat, gather_next):
        if gather_next:
            _gather(g + 1, 1 - b, sb)
        _wait_gather(b)
        if wait_scat:
            _wait_scatter(b)

        def _cvt(w):
            lo = lax.bitcast_convert_type(w << 16, jnp.float32)
            hi = lax.bitcast_convert_type(w & jnp.int32(-65536), jnp.float32)
            return lo, hi

        def _row(r, rc):
            for j in range(D // 32):
                eq0, eq1 = _cvt(qvv[b][r, pl.ds(j * 16, 16)])
                v0, v1 = _cvt(qvv[b][r, pl.ds(64 + j * 16, 16)])
                ek0 = ekv[b][r, pl.ds(j * 32, 16)]
                ek1 = ekv[b][r, pl.ds(j * 32 + 16, 16)]
                mv[b][r, pl.ds(j * 32, 16)] = v0 / (1.0 + ek0 * eq0)
                mv[b][r, pl.ds(j * 32 + 16, 16)] = v1 / (1.0 + ek1 * eq1)
            return rc
        lax.fori_loop(0, C, _row, 0)
        pltpu.async_copy(mv[b], agg.at[idxv[sb].at[1, g]], ssem[b], add=True)

    def _super(sc, sb, first):
        if not first:
            pltpu.make_async_copy(
                ei_hbm.at[0, wid, 0], idxv[sb].at[0], isem).wait()
            pltpu.make_async_copy(
                ei_hbm.at[1, wid, 0], idxv[sb].at[1], isem).wait()
            _wait_scatter(0)
            _wait_scatter(1)
        scn = jnp.where(sc + 1 >= SG, 0, sc + 1)
        pltpu.async_copy(ei_hbm.at[0, wid, scn], idxv[1 - sb].at[0], isem)
        pltpu.async_copy(ei_hbm.at[1, wid, scn], idxv[1 - sb].at[1], isem)
        _gather(0, 0, sb)
        _step(0, 0, sb, False, True)
        _step(1, 1, sb, False, True)

        def _pairs(p, cc):
            g = 2 * p
            _step(g, 0, sb, True, True)
            _step(g + 1, 1, sb, True, True)
            return cc
        lax.fori_loop(1, G // 2 - 1, _pairs, 0)
        _step(G - 2, 0, sb, True, True)
        _step(G - 1, 1, sb, True, False)

    pltpu.sync_copy(ei_hbm.at[0, wid, 0], idxv[0].at[0])
    pltpu.sync_copy(ei_hbm.at[1, wid, 0], idxv[0].at[1])
    _super(0, 0, True)

    def _souter(p, cc):
        _super(2 * p + 1, 1, False)
        _super(2 * p + 2, 0, False)
        return cc
    lax.fori_loop(0, (SG - 1) // 2, _souter, 0)
    pltpu.make_async_copy(ei_hbm.at[0, wid, 0], idxv[1].at[0], isem).wait()
    pltpu.make_async_copy(ei_hbm.at[1, wid, 0], idxv[1].at[1], isem).wait()
    _wait_scatter(0)
    _wait_scatter(1)

    plsc.subcore_barrier()
    pltpu.sync_copy(agg.at[pl.ds(s * RPT, RPT)],
                    out_hbm.at[c, pl.ds(s * RPT, RPT)])


def kernel(x, edge_index, Wk1, bk1, Wq1, bq1, Wv1, bv1, Ws1, b1,
           Wk2, bk2, Wq2, bq2, Wv2, bv2, Ws2, b2):
    ei = edge_index.reshape(2, NW, SG, G, C)

    perm = jnp.asarray(PERM)
    w1 = jnp.concatenate(
        [-Wk1, -Wq1[:, perm], Wv1[:, perm], Ws1], axis=1)
    b1c = jnp.broadcast_to(jnp.concatenate(
        [-bk1, -bq1[perm], bv1[perm], b1])[None, :], (8, 4 * D))
    w2 = jnp.concatenate(
        [-Wk2, -Wq2[:, perm], Wv2[:, perm], Ws2], axis=1)
    b2c = jnp.broadcast_to(jnp.concatenate(
        [-bk2, -bq2[perm], bv2[perm], b2])[None, :], (8, 4 * D))

    def _packqv(eq, v):
        pk = lambda t: lax.bitcast_convert_type(
            t.reshape(N, D // 2, 2), jnp.int32)
        return jnp.concatenate([pk(eq), pk(v)], axis=1)

    ek1, eq1, v1, s1 = _proj(x, w1, b1c)
    part1 = _edge_kernel(ek1, _packqv(eq1, v1), ei)
    ek2, eq2, v2, s2 = _relu_proj(part1, s1, w2, b2c)
    part2 = _edge_kernel(ek2, _packqv(eq2, v2), ei)
    return _final(part2, s2)


# speedup vs baseline: 6.2759x; 1.2381x over previous
import functools

import jax
import jax.numpy as jnp
from jax import lax
from jax.experimental import pallas as pl
from jax.experimental.pallas import tpu as pltpu
from jax.experimental.pallas import tpu_sc as plsc

N = 10000
E = 320000
D = 128

NC = 2
NS = 16
NW = NC * NS
EPW = E // NW
C = 40
NCHUNK = EPW // C
G = 10
SG = NCHUNK // G
NPAD = 10240
RPT = NPAD // NS

MBLK = 1000
GRID = N // MBLK


def _pack_cols(t):
    u = lax.bitcast_convert_type(t.astype(jnp.bfloat16), jnp.uint16)
    words = []
    for g in range(D // 32):
        lo = u[:, g * 32:g * 32 + 16].astype(jnp.int32)
        hi = u[:, g * 32 + 16:g * 32 + 32].astype(jnp.int32) << 16
        words.append(lo | hi)
    return jnp.concatenate(words, axis=1)


def _proj_body(x_ref, w_ref, b_ref, ek_ref, qv_ref, s_ref):
    y = jnp.dot(x_ref[...], w_ref[...], preferred_element_type=jnp.float32)
    y = y + b_ref[0:1, :]
    ek_ref[...] = jnp.exp(y[:, 0 * D:1 * D])
    qv_ref[...] = jnp.concatenate(
        [_pack_cols(jnp.exp(y[:, 1 * D:2 * D])),
         _pack_cols(y[:, 2 * D:3 * D])], axis=1)
    s_ref[...] = y[:, 3 * D:4 * D]


def _relu_proj_body(p_ref, s1_ref, w_ref, b_ref,
                    ek_ref, qv_ref, s_ref):
    h = jnp.maximum(p_ref[0] + p_ref[1] + s1_ref[...], 0.0)
    y = jnp.dot(h, w_ref[...], preferred_element_type=jnp.float32)
    y = y + b_ref[0:1, :]
    ek_ref[...] = jnp.exp(y[:, 0 * D:1 * D])
    qv_ref[...] = jnp.concatenate(
        [_pack_cols(jnp.exp(y[:, 1 * D:2 * D])),
         _pack_cols(y[:, 2 * D:3 * D])], axis=1)
    s_ref[...] = y[:, 3 * D:4 * D]


def _final_body(p_ref, s2_ref, o_ref):
    o_ref[...] = p_ref[0] + p_ref[1] + s2_ref[...]


_row_spec = pl.BlockSpec((MBLK, D), lambda i: (i, 0))
_part_spec = pl.BlockSpec((2, MBLK, D), lambda i: (0, i, 0))
_w_spec = pl.BlockSpec((D, 4 * D), lambda i: (0, 0))
_b_spec = pl.BlockSpec((8, 4 * D), lambda i: (0, 0))
_out3 = (jax.ShapeDtypeStruct((N, D), jnp.float32),
         jax.ShapeDtypeStruct((N, D), jnp.int32),
         jax.ShapeDtypeStruct((N, D), jnp.float32))

_proj = pl.pallas_call(
    _proj_body,
    grid=(GRID,),
    in_specs=[_row_spec, _w_spec, _b_spec],
    out_specs=(_row_spec,) * 3,
    out_shape=_out3,
)

_relu_proj = pl.pallas_call(
    _relu_proj_body,
    grid=(GRID,),
    in_specs=[_part_spec, _row_spec, _w_spec, _b_spec],
    out_specs=(_row_spec,) * 3,
    out_shape=_out3,
)

_final = pl.pallas_call(
    _final_body,
    grid=(GRID,),
    in_specs=[_part_spec, _row_spec],
    out_specs=_row_spec,
    out_shape=jax.ShapeDtypeStruct((N, D), jnp.float32),
)


_mesh = plsc.VectorSubcoreMesh(
    core_axis_name="c", subcore_axis_name="s", num_cores=NC, num_subcores=NS)


@functools.partial(
    pl.kernel,
    out_type=jax.ShapeDtypeStruct((NC, NPAD, D), jnp.float32),
    mesh=_mesh,
    scratch_types=[
        [pltpu.VMEM((2, G, C), jnp.int32)] * 2,
        [pltpu.VMEM((C, D), jnp.float32)] * 2,
        [pltpu.VMEM((C, D), jnp.int32)] * 2,
        [pltpu.VMEM((C, D), jnp.float32)] * 2,
        pltpu.VMEM_SHARED((NPAD, D), jnp.float32),
        [pltpu.SemaphoreType.DMA] * 2,
        [pltpu.SemaphoreType.DMA] * 2,
        pltpu.SemaphoreType.DMA,
    ],
)
def _edge_kernel(ek_hbm, qv_hbm, ei_hbm, out_hbm,
                 idxv, ekv, qvv, mv, agg, gsem, ssem, isem):
    c = lax.axis_index("c")
    s = lax.axis_index("s")
    wid = s * NC + c

    def _zero_row(r, carry):
        for j in range(D // 16):
            mv[0][r, pl.ds(j * 16, 16)] = jnp.zeros((16,), jnp.float32)
        return carry
    lax.fori_loop(0, C, _zero_row, 0)
    for t in range(RPT // C):
        pltpu.async_copy(mv[0], agg.at[pl.ds(s * RPT + t * C, C)], gsem[0])
    for t in range(RPT // C):
        pltpu.make_async_copy(mv[0], agg.at[pl.ds(s * RPT, C)], gsem[0]).wait()
    plsc.subcore_barrier()

    def _gather(g, b, sb):
        pltpu.async_copy(ek_hbm.at[idxv[sb].at[1, g]], ekv[b], gsem[b])
        pltpu.async_copy(qv_hbm.at[idxv[sb].at[0, g]], qvv[b], gsem[b])

    def _wait_gather(b):
        i0 = idxv[0].at[0, 0]
        pltpu.make_async_copy(ek_hbm.at[i0], ekv[b], gsem[b]).wait()
        pltpu.make_async_copy(qv_hbm.at[i0], qvv[b], gsem[b]).wait()

    def _wait_scatter(b):
        pltpu.make_async_copy(mv[b], agg.at[idxv[0].at[1, 0]], ssem[b]).wait()

    def _step(g, b, sb, wait_scat, gather_next):
        if gather_next:
            _gather(g + 1, 1 - b, sb)
        _wait_gather(b)
        if wait_scat:
            _wait_scatter(b)

        def _cvt(w):
            lo = lax.bitcast_convert_type(w << 16, jnp.float32)
            hi = lax.bitcast_convert_type(w & jnp.int32(-65536), jnp.float32)
            return lo, hi

        def _row(r, rc):
            for j in range(D // 32):
                eq0, eq1 = _cvt(qvv[b][r, pl.ds(j * 16, 16)])
                v0, v1 = _cvt(qvv[b][r, pl.ds(64 + j * 16, 16)])
                ek0 = ekv[b][r, pl.ds(j * 32, 16)]
                ek1 = ekv[b][r, pl.ds(j * 32 + 16, 16)]
                mv[b][r, pl.ds(j * 32, 16)] = v0 / (1.0 + ek0 * eq0)
                mv[b][r, pl.ds(j * 32 + 16, 16)] = v1 / (1.0 + ek1 * eq1)
            return rc
        lax.fori_loop(0, C, _row, 0)
        pltpu.async_copy(mv[b], agg.at[idxv[sb].at[1, g]], ssem[b], add=True)

    def _super(sc, sb, first):
        if not first:
            pltpu.make_async_copy(
                ei_hbm.at[0, wid, 0], idxv[sb].at[0], isem).wait()
            pltpu.make_async_copy(
                ei_hbm.at[1, wid, 0], idxv[sb].at[1], isem).wait()
            _wait_scatter(0)
            _wait_scatter(1)
        scn = jnp.where(sc + 1 >= SG, 0, sc + 1)
        pltpu.async_copy(ei_hbm.at[0, wid, scn], idxv[1 - sb].at[0], isem)
        pltpu.async_copy(ei_hbm.at[1, wid, scn], idxv[1 - sb].at[1], isem)
        _gather(0, 0, sb)
        _step(0, 0, sb, False, True)
        _step(1, 1, sb, False, True)

        def _pairs(p, cc):
            g = 2 * p
            _step(g, 0, sb, True, True)
            _step(g + 1, 1, sb, True, True)
            return cc
        lax.fori_loop(1, G // 2 - 1, _pairs, 0)
        _step(G - 2, 0, sb, True, True)
        _step(G - 1, 1, sb, True, False)

    pltpu.sync_copy(ei_hbm.at[0, wid, 0], idxv[0].at[0])
    pltpu.sync_copy(ei_hbm.at[1, wid, 0], idxv[0].at[1])
    _super(0, 0, True)

    def _souter(p, cc):
        _super(2 * p + 1, 1, False)
        _super(2 * p + 2, 0, False)
        return cc
    lax.fori_loop(0, (SG - 1) // 2, _souter, 0)
    pltpu.make_async_copy(ei_hbm.at[0, wid, 0], idxv[1].at[0], isem).wait()
    pltpu.make_async_copy(ei_hbm.at[1, wid, 0], idxv[1].at[1], isem).wait()
    _wait_scatter(0)
    _wait_scatter(1)

    plsc.subcore_barrier()
    pltpu.sync_copy(agg.at[pl.ds(s * RPT, RPT)],
                    out_hbm.at[c, pl.ds(s * RPT, RPT)])


def kernel(x, edge_index, Wk1, bk1, Wq1, bq1, Wv1, bv1, Ws1, b1,
           Wk2, bk2, Wq2, bq2, Wv2, bv2, Ws2, b2):
    ei = edge_index.reshape(2, NW, SG, G, C)

    w1 = jnp.concatenate([-Wk1, -Wq1, Wv1, Ws1], axis=1)
    b1c = jnp.broadcast_to(jnp.concatenate(
        [-bk1, -bq1, bv1, b1])[None, :], (8, 4 * D))
    w2 = jnp.concatenate([-Wk2, -Wq2, Wv2, Ws2], axis=1)
    b2c = jnp.broadcast_to(jnp.concatenate(
        [-bk2, -bq2, bv2, b2])[None, :], (8, 4 * D))

    ek1, qv1, s1 = _proj(x, w1, b1c)
    part1 = _edge_kernel(ek1, qv1, ei)
    ek2, qv2, s2 = _relu_proj(part1, s1, w2, b2c)
    part2 = _edge_kernel(ek2, qv2, ei)
    return _final(part2, s2)


# speedup vs baseline: 6.2816x; 1.0009x over previous
import functools

import jax
import jax.numpy as jnp
from jax import lax
from jax.experimental import pallas as pl
from jax.experimental.pallas import tpu as pltpu
from jax.experimental.pallas import tpu_sc as plsc

N = 10000
E = 320000
D = 128

NC = 2
NS = 16
NW = NC * NS
EPW = E // NW
C = 40
NCHUNK = EPW // C
G = 10
SG = NCHUNK // G
NPAD = 10240
RPT = NPAD // NS

MBLK = 1000
GRID = N // MBLK


def _pack_cols(t):
    u = lax.bitcast_convert_type(t.astype(jnp.bfloat16), jnp.uint16)
    words = []
    for g in range(D // 32):
        lo = u[:, g * 32:g * 32 + 16].astype(jnp.int32)
        hi = u[:, g * 32 + 16:g * 32 + 32].astype(jnp.int32) << 16
        words.append(lo | hi)
    return jnp.concatenate(words, axis=1)


def _proj_body(x_ref, w_ref, b_ref, ek_ref, qv_ref, s_ref):
    y = jnp.dot(x_ref[...], w_ref[...], preferred_element_type=jnp.float32)
    y = y + b_ref[0:1, :]
    ek_ref[...] = jnp.exp(y[:, 0 * D:1 * D])
    qv_ref[...] = jnp.concatenate(
        [_pack_cols(jnp.exp(y[:, 1 * D:2 * D])),
         _pack_cols(y[:, 2 * D:3 * D])], axis=1)
    s_ref[...] = y[:, 3 * D:4 * D]


def _relu_proj_body(p_ref, s1_ref, w_ref, b_ref,
                    ek_ref, qv_ref, s_ref):
    h = jnp.maximum(p_ref[0] + p_ref[1] + s1_ref[...], 0.0)
    y = jnp.dot(h, w_ref[...], preferred_element_type=jnp.float32)
    y = y + b_ref[0:1, :]
    ek_ref[...] = jnp.exp(y[:, 0 * D:1 * D])
    qv_ref[...] = jnp.concatenate(
        [_pack_cols(jnp.exp(y[:, 1 * D:2 * D])),
         _pack_cols(y[:, 2 * D:3 * D])], axis=1)
    s_ref[...] = y[:, 3 * D:4 * D]


def _final_body(p_ref, s2_ref, o_ref):
    o_ref[...] = p_ref[0] + p_ref[1] + s2_ref[...]


_row_spec = pl.BlockSpec((MBLK, D), lambda i: (i, 0))
_part_spec = pl.BlockSpec((2, MBLK, D), lambda i: (0, i, 0))
_w_spec = pl.BlockSpec((D, 4 * D), lambda i: (0, 0))
_b_spec = pl.BlockSpec((8, 4 * D), lambda i: (0, 0))
_out3 = (jax.ShapeDtypeStruct((N, D), jnp.float32),
         jax.ShapeDtypeStruct((N, D), jnp.int32),
         jax.ShapeDtypeStruct((N, D), jnp.float32))

_proj = pl.pallas_call(
    _proj_body,
    grid=(GRID,),
    in_specs=[_row_spec, _w_spec, _b_spec],
    out_specs=(_row_spec,) * 3,
    out_shape=_out3,
)

_relu_proj = pl.pallas_call(
    _relu_proj_body,
    grid=(GRID,),
    in_specs=[_part_spec, _row_spec, _w_spec, _b_spec],
    out_specs=(_row_spec,) * 3,
    out_shape=_out3,
)

_final = pl.pallas_call(
    _final_body,
    grid=(GRID,),
    in_specs=[_part_spec, _row_spec],
    out_specs=_row_spec,
    out_shape=jax.ShapeDtypeStruct((N, D), jnp.float32),
)


_mesh = plsc.VectorSubcoreMesh(
    core_axis_name="c", subcore_axis_name="s", num_cores=NC, num_subcores=NS)


@functools.partial(
    pl.kernel,
    out_type=jax.ShapeDtypeStruct((NC, NPAD, D), jnp.float32),
    mesh=_mesh,
    scratch_types=[
        [pltpu.VMEM((2, G, C), jnp.int32)] * 2,
        [pltpu.VMEM((C, D), jnp.float32)] * 2,
        [pltpu.VMEM((C, D), jnp.int32)] * 2,
        [pltpu.VMEM((C, D), jnp.float32)] * 2,
        pltpu.VMEM_SHARED((NPAD, D), jnp.float32),
        [pltpu.SemaphoreType.DMA] * 2,
        [pltpu.SemaphoreType.DMA] * 2,
        pltpu.SemaphoreType.DMA,
    ],
)
def _edge_kernel(ek_hbm, qv_hbm, ei_hbm, out_hbm,
                 idxv, ekv, qvv, mv, agg, gsem, ssem, isem):
    c = lax.axis_index("c")
    s = lax.axis_index("s")
    wid = s * NC + c

    def _gather(g, b, sb):
        pltpu.async_copy(ek_hbm.at[idxv[sb].at[1, g]], ekv[b], gsem[b])
        pltpu.async_copy(qv_hbm.at[idxv[sb].at[0, g]], qvv[b], gsem[b])

    def _wait_gather(b):
        i0 = idxv[0].at[0, 0]
        pltpu.make_async_copy(ek_hbm.at[i0], ekv[b], gsem[b]).wait()
        pltpu.make_async_copy(qv_hbm.at[i0], qvv[b], gsem[b]).wait()

    def _wait_scatter(b):
        pltpu.make_async_copy(mv[b], agg.at[idxv[0].at[1, 0]], ssem[b]).wait()

    def _step(g, b, sb, wait_scat, gather_next):
        if gather_next:
            _gather(g + 1, 1 - b, sb)
        _wait_gather(b)
        if wait_scat:
            _wait_scatter(b)

        def _cvt(w):
            lo = lax.bitcast_convert_type(w << 16, jnp.float32)
            hi = lax.bitcast_convert_type(w & jnp.int32(-65536), jnp.float32)
            return lo, hi

        def _row(r, rc):
            for j in range(D // 32):
                eq0, eq1 = _cvt(qvv[b][r, pl.ds(j * 16, 16)])
                v0, v1 = _cvt(qvv[b][r, pl.ds(64 + j * 16, 16)])
                ek0 = ekv[b][r, pl.ds(j * 32, 16)]
                ek1 = ekv[b][r, pl.ds(j * 32 + 16, 16)]
                mv[b][r, pl.ds(j * 32, 16)] = v0 / (1.0 + ek0 * eq0)
                mv[b][r, pl.ds(j * 32 + 16, 16)] = v1 / (1.0 + ek1 * eq1)
            return rc
        lax.fori_loop(0, C, _row, 0)
        pltpu.async_copy(mv[b], agg.at[idxv[sb].at[1, g]], ssem[b], add=True)

    def _super(sc, sb, first):
        if not first:
            pltpu.make_async_copy(
                ei_hbm.at[0, wid, 0], idxv[sb].at[0], isem).wait()
            pltpu.make_async_copy(
                ei_hbm.at[1, wid, 0], idxv[sb].at[1], isem).wait()
            _wait_scatter(0)
            _wait_scatter(1)
        scn = jnp.where(sc + 1 >= SG, 0, sc + 1)
        pltpu.async_copy(ei_hbm.at[0, wid, scn], idxv[1 - sb].at[0], isem)
        pltpu.async_copy(ei_hbm.at[1, wid, scn], idxv[1 - sb].at[1], isem)
        if not first:
            _gather(0, 0, sb)
        _step(0, 0, sb, False, True)
        _step(1, 1, sb, False, True)

        def _pairs(p, cc):
            g = 2 * p
            _step(g, 0, sb, True, True)
            _step(g + 1, 1, sb, True, True)
            return cc
        lax.fori_loop(1, G // 2 - 1, _pairs, 0)
        _step(G - 2, 0, sb, True, True)
        _step(G - 1, 1, sb, True, False)

    pltpu.sync_copy(ei_hbm.at[0, wid, 0], idxv[0].at[0])
    pltpu.sync_copy(ei_hbm.at[1, wid, 0], idxv[0].at[1])
    _gather(0, 0, 0)

    def _zero_row(r, carry):
        for j in range(D // 16):
            mv[0][r, pl.ds(j * 16, 16)] = jnp.zeros((16,), jnp.float32)
        return carry
    lax.fori_loop(0, C, _zero_row, 0)
    for t in range(RPT // C):
        pltpu.async_copy(mv[0], agg.at[pl.ds(s * RPT + t * C, C)], ssem[0])
    for t in range(RPT // C):
        pltpu.make_async_copy(mv[0], agg.at[pl.ds(s * RPT, C)], ssem[0]).wait()
    plsc.subcore_barrier()

    _super(0, 0, True)

    def _souter(p, cc):
        _super(2 * p + 1, 1, False)
        _super(2 * p + 2, 0, False)
        return cc
    lax.fori_loop(0, (SG - 1) // 2, _souter, 0)
    pltpu.make_async_copy(ei_hbm.at[0, wid, 0], idxv[1].at[0], isem).wait()
    pltpu.make_async_copy(ei_hbm.at[1, wid, 0], idxv[1].at[1], isem).wait()
    _wait_scatter(0)
    _wait_scatter(1)

    plsc.subcore_barrier()
    pltpu.sync_copy(agg.at[pl.ds(s * RPT, RPT)],
                    out_hbm.at[c, pl.ds(s * RPT, RPT)])


def kernel(x, edge_index, Wk1, bk1, Wq1, bq1, Wv1, bv1, Ws1, b1,
           Wk2, bk2, Wq2, bq2, Wv2, bv2, Ws2, b2):
    ei = edge_index.reshape(2, NW, SG, G, C)

    w1 = jnp.concatenate([-Wk1, -Wq1, Wv1, Ws1], axis=1)
    b1c = jnp.broadcast_to(jnp.concatenate(
        [-bk1, -bq1, bv1, b1])[None, :], (8, 4 * D))
    w2 = jnp.concatenate([-Wk2, -Wq2, Wv2, Ws2], axis=1)
    b2c = jnp.broadcast_to(jnp.concatenate(
        [-bk2, -bq2, bv2, b2])[None, :], (8, 4 * D))

    ek1, qv1, s1 = _proj(x, w1, b1c)
    part1 = _edge_kernel(ek1, qv1, ei)
    ek2, qv2, s2 = _relu_proj(part1, s1, w2, b2c)
    part2 = _edge_kernel(ek2, qv2, ei)
    return _final(part2, s2)
